# baseline (speedup 1.0000x reference)
import functools

import jax
import jax.numpy as jnp
from jax import lax
from jax.experimental import pallas as pl
from jax.experimental.pallas import tpu as pltpu
from jax.experimental.pallas import tpu_sc as plsc

N = 10000
E = 320000
D = 128
G = 64
NP = 10240
NB = NP // 128
NR = 10368
RB = 512
NG = NP // RB
NSC = 2
NSUB = 16
NW = NSC * NSUB
KE = 64
EP = 321536
EW = EP // NW
NEG = -3.0e38


def _leaky(x, s):
    return jnp.where(x > 0, x, s * x)


def _eye128():
    a = lax.broadcasted_iota(jnp.int32, (128, 128), 0)
    b = lax.broadcasted_iota(jnp.int32, (128, 128), 1)
    return (a == b).astype(jnp.float32)


def _k_gat_pre(x_ref, a_ref, wl_ref, bl_ref, wr_ref, br_ref, att_ref,
               xl2_ref, xr_ref, exs_ref):
    x = x_ref[...]
    att = att_ref[...]
    xl = jnp.dot(x, wl_ref[...], preferred_element_type=jnp.float32) + bl_ref[...]
    xr = jnp.dot(x, wr_ref[...], preferred_element_type=jnp.float32) + br_ref[...]
    xl2_ref[:, :D] = xl
    lane0 = (lax.broadcasted_iota(jnp.int32, (1, D), 1) == 0).astype(jnp.float32)
    al = jnp.sum(xl * att, axis=1, keepdims=True)
    bias = (a_ref[...] - 1.0) * 1e4 + 0.6 * al
    xl2_ref[:, D:] = bias * lane0
    xr_ref[...] = xr
    t = xl + xr
    s = 0.6 * al + 0.4 * jnp.sum(jnp.abs(t) * att, axis=1, keepdims=True)
    exs_ref[...] = jnp.broadcast_to(jnp.exp(s), (RB, D))


def _gat_pre(x, alive, p):
    f = pl.pallas_call(
        _k_gat_pre,
        grid=(NG,),
        in_specs=[pl.BlockSpec((RB, D), lambda i: (i, 0)),
                  pl.BlockSpec((RB, 1), lambda i: (i, 0)),
                  pl.BlockSpec((D, D), lambda i: (0, 0)),
                  pl.BlockSpec((1, D), lambda i: (0, 0)),
                  pl.BlockSpec((D, D), lambda i: (0, 0)),
                  pl.BlockSpec((1, D), lambda i: (0, 0)),
                  pl.BlockSpec((1, D), lambda i: (0, 0))],
        out_specs=[pl.BlockSpec((RB, 2 * D), lambda i: (i, 0)),
                   pl.BlockSpec((RB, D), lambda i: (i, 0)),
                   pl.BlockSpec((RB, D), lambda i: (i, 0))],
        out_shape=[jax.ShapeDtypeStruct((NP, 2 * D), jnp.float32),
                   jax.ShapeDtypeStruct((NP, D), jnp.float32),
                   jax.ShapeDtypeStruct((NP, D), jnp.float32)],
    )
    return f(x, alive, p['Wl'], p['bl'].reshape(1, D), p['Wr'],
             p['br'].reshape(1, D), p['att'].reshape(1, D))


def _k_gat_post(a0_ref, a1_ref, d0_ref, d1_ref, xl_ref, exs_ref, b_ref,
                al_ref, o_ref):
    num = a0_ref[...] + a1_ref[...]
    den = d0_ref[...] + d1_ref[...]
    exs = exs_ref[:, 0:1]
    num = num + exs * xl_ref[...]
    den = den + exs
    o = num / (den + 1e-16) + b_ref[...]
    o_ref[...] = _leaky(o, 0.01) * al_ref[...]


def _gat_post(a0, a1, d0, d1, xl, exs, b, alive):
    f = pl.pallas_call(
        _k_gat_post,
        grid=(NG,),
        in_specs=[pl.BlockSpec((RB, D), lambda i: (i, 0)),
                  pl.BlockSpec((RB, D), lambda i: (i, 0)),
                  pl.BlockSpec((RB, 1), lambda i: (i, 0)),
                  pl.BlockSpec((RB, 1), lambda i: (i, 0)),
                  pl.BlockSpec((RB, D), lambda i: (i, 0)),
                  pl.BlockSpec((RB, D), lambda i: (i, 0)),
                  pl.BlockSpec((1, D), lambda i: (0, 0)),
                  pl.BlockSpec((RB, 1), lambda i: (i, 0))],
        out_specs=pl.BlockSpec((RB, D), lambda i: (i, 0)),
        out_shape=jax.ShapeDtypeStruct((NP, D), jnp.float32),
    )
    return f(a0, a1, d0, d1, xl, exs, b.reshape(1, D), alive)


def _k_sage_pre(x_ref, a_ref, o_ref):
    o_ref[:, :D] = x_ref[...]
    lane0 = (lax.broadcasted_iota(jnp.int32, (1, D), 1) == 0).astype(jnp.float32)
    o_ref[:, D:] = a_ref[...] * lane0


def _sage_pre(x, alive):
    f = pl.pallas_call(
        _k_sage_pre,
        grid=(NG,),
        in_specs=[pl.BlockSpec((RB, D), lambda i: (i, 0)),
                  pl.BlockSpec((RB, 1), lambda i: (i, 0))],
        out_specs=pl.BlockSpec((RB, 2 * D), lambda i: (i, 0)),
        out_shape=jax.ShapeDtypeStruct((NP, 2 * D), jnp.float32),
    )
    return f(x, alive)


def _k_sage_post(a0_ref, a1_ref, d0_ref, d1_ref, x_ref, wl_ref, wr_ref,
                 bl_ref, al_ref, o_ref):
    num = a0_ref[...] + a1_ref[...]
    cnt = d0_ref[...] + d1_ref[...]
    mean = num / jnp.maximum(cnt, 1.0)
    o = (jnp.dot(mean, wl_ref[...], preferred_element_type=jnp.float32)
         + jnp.dot(x_ref[...], wr_ref[...], preferred_element_type=jnp.float32)
         + bl_ref[...])
    o_ref[...] = _leaky(o, 0.01) * al_ref[...]


def _sage_post(a0, a1, d0, d1, x, p, alive):
    f = pl.pallas_call(
        _k_sage_post,
        grid=(NG,),
        in_specs=[pl.BlockSpec((RB, D), lambda i: (i, 0)),
                  pl.BlockSpec((RB, D), lambda i: (i, 0)),
                  pl.BlockSpec((RB, 1), lambda i: (i, 0)),
                  pl.BlockSpec((RB, 1), lambda i: (i, 0)),
                  pl.BlockSpec((RB, D), lambda i: (i, 0)),
                  pl.BlockSpec((D, D), lambda i: (0, 0)),
                  pl.BlockSpec((D, D), lambda i: (0, 0)),
                  pl.BlockSpec((1, D), lambda i: (0, 0)),
                  pl.BlockSpec((RB, 1), lambda i: (i, 0))],
        out_specs=pl.BlockSpec((RB, D), lambda i: (i, 0)),
        out_shape=jax.ShapeDtypeStruct((NP, D), jnp.float32),
    )
    return f(a0, a1, d0, d1, x, p['Wl'], p['Wr'], p['bl'].reshape(1, D), alive)


def _k_score_tables(x_ref, w_ref, bcol_ref, acol_ref, score_ref, tab_ref):
    w = w_ref[...]
    rnorm = 1.0 / (jnp.sqrt(jnp.sum(w * w)) + 1e-16)
    g_row = lax.broadcasted_iota(jnp.int32, (1, 128), 1).astype(jnp.float32)
    counts_a = jnp.zeros((1, 128), jnp.float32)
    counts_o = jnp.zeros((1, 128), jnp.float32)
    starts = jnp.zeros((1, 128), jnp.float32)
    for r in range(NB):
        rows = x_ref[pl.ds(r * 128, 128), :]
        s = jnp.sum(rows * w, axis=1, keepdims=True)
        score_ref[pl.ds(r * 128, 128), :] = jnp.tanh(s * rnorm)
        bcol = bcol_ref[pl.ds(r * 128, 128), :].astype(jnp.float32)
        acol = acol_ref[pl.ds(r * 128, 128), :]
        eqg = (bcol == g_row).astype(jnp.float32)
        counts_a = counts_a + jnp.sum(eqg * acol, axis=0, keepdims=True)
        counts_o = counts_o + jnp.sum(eqg, axis=0, keepdims=True)
        starts = starts + jnp.sum((bcol < g_row).astype(jnp.float32),
                                  axis=0, keepdims=True)
    kk = jnp.where(g_row < G, jnp.floor((counts_a + 1.0) * 0.5), 0.0)
    lt = (lax.broadcasted_iota(jnp.int32, (128, 128), 0)
          < lax.broadcasted_iota(jnp.int32, (128, 128), 1)).astype(jnp.float32)
    prefix_kk = jnp.dot(kk, lt, preferred_element_type=jnp.float32)
    tab_ref[...] = jnp.concatenate(
        [kk, starts, counts_o, prefix_kk,
         jnp.zeros((4, 128), jnp.float32)], axis=0)


def _score_tables(x, w, bcol, acol):
    f = pl.pallas_call(
        _k_score_tables,
        grid=(1,),
        in_specs=[pl.BlockSpec((NP, D), lambda i: (0, 0)),
                  pl.BlockSpec((1, D), lambda i: (0, 0)),
                  pl.BlockSpec((NP, 1), lambda i: (0, 0)),
                  pl.BlockSpec((NP, 1), lambda i: (0, 0))],
        out_specs=[pl.BlockSpec((NP, 1), lambda i: (0, 0)),
                   pl.BlockSpec((8, 128), lambda i: (0, 0))],
        out_shape=[jax.ShapeDtypeStruct((NP, 1), jnp.float32),
                   jax.ShapeDtypeStruct((8, 128), jnp.float32)],
    )
    return f(x, w.reshape(1, D), bcol, acol)


def _k_rank(score_ref, bcol_ref, acol_ref, ocol_ref, tab_ref, tabs_ref,
            bsm_ref, tsc_ref, al_ref, ord_ref):
    r = pl.program_id(0)
    eye = _eye128()
    g_col = lax.broadcasted_iota(jnp.int32, (128, 1), 0).astype(jnp.float32)
    scol = score_ref[pl.ds(r * 128, 128), :]
    bcolf = bcol_ref[pl.ds(r * 128, 128), :].astype(jnp.float32)
    acolf = acol_ref[pl.ds(r * 128, 128), :]
    ocolf = ocol_ref[pl.ds(r * 128, 128), :]
    si = jnp.sum(eye * scol, axis=0, keepdims=True)
    bi = jnp.sum(eye * bcolf, axis=0, keepdims=True)
    ai = jnp.sum(eye * acolf, axis=0, keepdims=True)
    oi = jnp.sum(eye * ocolf, axis=0, keepdims=True)

    bmin = bsm_ref[r, 0]
    bmax = bsm_ref[r, 127]
    c_lo = tabs_ref[1, bmin].astype(jnp.int32)
    c_hi = (tabs_ref[1, bmax] + tabs_ref[2, bmax]).astype(jnp.int32)
    t_lo = c_lo // 128
    t_hi = (c_hi + 127) // 128

    def body(t, rank):
        sj = score_ref[pl.ds(t * 128, 128), :]
        bj = bcol_ref[pl.ds(t * 128, 128), :].astype(jnp.float32)
        aj = acol_ref[pl.ds(t * 128, 128), :]
        oj = ocol_ref[pl.ds(t * 128, 128), :]
        beats = (sj > si) | ((sj == si) & (oj < oi))
        cmp = (bj == bi) & beats & (aj > 0)
        return rank + jnp.sum(cmp.astype(jnp.float32), axis=0, keepdims=True)

    rank = lax.fori_loop(t_lo, t_hi, body, jnp.zeros((1, 128), jnp.float32))

    kk_col = jnp.sum(eye * tab_ref[0:1, :], axis=1, keepdims=True)
    pf_col = jnp.sum(eye * tab_ref[3:4, :], axis=1, keepdims=True)
    bmask = (g_col == bi).astype(jnp.float32)
    kk_b = jnp.sum(bmask * kk_col, axis=0, keepdims=True)
    pf_b = jnp.sum(bmask * pf_col, axis=0, keepdims=True)
    kept = (ai > 0) & (rank < kk_b)
    tsc_ref[...] = jnp.where(kept, si, 0.0).reshape(1, 1, 128)
    al_ref[...] = jnp.where(kept, 1.0, 0.0).reshape(1, 1, 128)
    ord_ref[...] = jnp.where(kept, pf_b + rank, 0.0).reshape(1, 1, 128)


def _rank(score, bcol, acol, ocol, tab, bsm):
    f = pl.pallas_call(
        _k_rank,
        grid=(NB,),
        in_specs=[pl.BlockSpec((NP, 1), lambda i: (0, 0)),
                  pl.BlockSpec((NP, 1), lambda i: (0, 0)),
                  pl.BlockSpec((NP, 1), lambda i: (0, 0)),
                  pl.BlockSpec((NP, 1), lambda i: (0, 0)),
                  pl.BlockSpec((8, 128), lambda i: (0, 0)),
                  pl.BlockSpec(memory_space=pltpu.SMEM),
                  pl.BlockSpec(memory_space=pltpu.SMEM)],
        out_specs=[pl.BlockSpec((1, 1, 128), lambda i: (i, 0, 0))] * 3,
        out_shape=[jax.ShapeDtypeStruct((NB, 1, 128), jnp.float32)] * 3,
    )
    return f(score, bcol, acol, ocol, tab, tab, bsm)


def _k_scale_rows(x_ref, t_ref, o_ref):
    o_ref[...] = x_ref[...] * t_ref[...]


def _scale_rows(x, tsc):
    f = pl.pallas_call(
        _k_scale_rows,
        grid=(NG,),
        in_specs=[pl.BlockSpec((RB, D), lambda i: (i, 0)),
                  pl.BlockSpec((RB, 1), lambda i: (i, 0))],
        out_specs=pl.BlockSpec((RB, D), lambda i: (i, 0)),
        out_shape=jax.ShapeDtypeStruct((NP, D), jnp.float32),
    )
    return f(x, tsc)


def _k_gpool(x_ref, acol_ref, tabs_ref, o_ref):
    pid = pl.program_id(0)
    sub = lax.broadcasted_iota(jnp.int32, (8, 1), 0).astype(jnp.float32)
    for gg in range(8):
        g = pid * 8 + gg
        kkg = tabs_ref[0, g].astype(jnp.int32)
        s = tabs_ref[1, g].astype(jnp.int32)
        co = tabs_ref[2, g].astype(jnp.int32)
        nch = (co + 7) // 8

        def body(t, carry):
            msum, mmax = carry
            rows = x_ref[pl.ds(s + t * 8, 8), :]
            am = acol_ref[pl.ds(s + t * 8, 8), :]
            pos = (s + t * 8).astype(jnp.float32) + sub
            rmask = (pos < (s + co).astype(jnp.float32)) & (am > 0)
            msum = msum + jnp.where(rmask, rows, 0.0)
            mmax = jnp.maximum(mmax, jnp.where(rmask, rows, NEG))
            return msum, mmax

        msum, mmax = lax.fori_loop(
            0, nch, body,
            (jnp.zeros((8, D), jnp.float32), jnp.full((8, D), NEG, jnp.float32)))
        colsum = jnp.sum(msum, axis=0, keepdims=True)
        colmax = jnp.max(mmax, axis=0, keepdims=True)
        kf = kkg.astype(jnp.float32)
        mean = colsum / jnp.maximum(kf, 1.0)
        mx = jnp.where(kkg > 0, colmax, 0.0)
        o_ref[pl.ds(gg, 1), 0:D] = mx
        o_ref[pl.ds(gg, 1), D:2 * D] = mean


def _gpool(x, acol, tab):
    f = pl.pallas_call(
        _k_gpool,
        grid=(8,),
        in_specs=[pl.BlockSpec((NP, D), lambda i: (0, 0)),
                  pl.BlockSpec((NP, 1), lambda i: (0, 0)),
                  pl.BlockSpec(memory_space=pltpu.SMEM)],
        out_specs=pl.BlockSpec((8, 2 * D), lambda i: (i, 0)),
        out_shape=jax.ShapeDtypeStruct((G, 2 * D), jnp.float32),
    )
    return f(x, acol, tab)


def _k_mlp(s1, s2, s3, t1, t2, t3, w1a, w1b, b1, w2, b2, w3, b3, o_ref):
    s = s1[...] + s2[...] + s3[...]
    t = t1[...] + t2[...] + t3[...]
    h = (jnp.dot(s, w1a[...], preferred_element_type=jnp.float32)
         + jnp.dot(t, w1b[...], preferred_element_type=jnp.float32) + b1[...])
    h = _leaky(h, 0.01)
    h = _leaky(jnp.dot(h, w2[...], preferred_element_type=jnp.float32) + b2[...], 0.01)
    lg = jnp.dot(h, w3[...], preferred_element_type=jnp.float32) + b3[...]
    lanemask = lax.broadcasted_iota(jnp.int32, (1, 128), 1) < 2
    ex = jnp.where(lanemask, jnp.exp(lg), 0.0)
    ssum = jnp.sum(ex, axis=1, keepdims=True)
    o_ref[...] = lg - jnp.log(ssum)


def _mlp(s1, s2, s3, t1, t2, t3, P):
    w1 = P['lin1']['W']
    w2 = jnp.pad(P['lin2']['W'], ((0, 0), (0, 64)))
    b2 = jnp.pad(P['lin2']['b'], (0, 64)).reshape(1, 128)
    w3 = jnp.pad(P['lin3']['W'], ((0, 64), (0, 126)))
    b3 = jnp.pad(P['lin3']['b'], (0, 126)).reshape(1, 128)
    f = pl.pallas_call(
        _k_mlp,
        grid=(1,),
        in_specs=[pl.BlockSpec((G, 2 * D), lambda i: (0, 0))] * 6
        + [pl.BlockSpec((2 * D, D), lambda i: (0, 0)),
           pl.BlockSpec((2 * D, D), lambda i: (0, 0)),
           pl.BlockSpec((1, D), lambda i: (0, 0)),
           pl.BlockSpec((D, D), lambda i: (0, 0)),
           pl.BlockSpec((1, D), lambda i: (0, 0)),
           pl.BlockSpec((D, D), lambda i: (0, 0)),
           pl.BlockSpec((1, D), lambda i: (0, 0))],
        out_specs=pl.BlockSpec((G, D), lambda i: (0, 0)),
        out_shape=jax.ShapeDtypeStruct((G, D), jnp.float32),
    )
    return f(s1, s2, s3, t1, t2, t3, w1[:256], w1[256:],
             P['lin1']['b'].reshape(1, D), w2, b2, w3, b3)


def _sum16(v):
    for b in range(4):
        idx = lax.iota(jnp.int32, 16) ^ (1 << b)
        v = v + v.at[idx].get(mode='promise_in_bounds')
    return v


def _sc_mesh():
    return plsc.VectorSubcoreMesh(core_axis_name="c", subcore_axis_name="s",
                                  num_cores=NSC, num_subcores=NSUB)


def _sc_edge_gat(xl2, xr, att, sd, z):
    @functools.partial(
        pl.kernel, mesh=_sc_mesh(),
        out_type=jax.ShapeDtypeStruct((NSC, NR, D), jnp.float32),
        scratch_types=[pltpu.VMEM_SHARED((NR, D), jnp.float32),
                       pltpu.VMEM((2 * KE,), jnp.int32),
                       pltpu.VMEM((2 * KE,), jnp.int32),
                       pltpu.VMEM((KE + 16,), jnp.int32),
                       pltpu.VMEM((KE, 2 * D), jnp.float32),
                       pltpu.VMEM((KE, D), jnp.float32),
                       pltpu.VMEM((D,), jnp.float32),
                       pltpu.VMEM((2 * KE, D), jnp.float32),
                       pltpu.VMEM((KE + 16,), jnp.float32),
                       pltpu.SemaphoreType.DMA],
    )
    def k(xl2_h, xr_h, att_h, sd_h, z_h, acc_o,
          accum, sdv, scidx, didxe, lrows, rrows, attv, srow2, exv, sem):
        cid = lax.axis_index("c")
        sid = lax.axis_index("s")
        wid = cid * NSUB + sid
        rows0 = sid * (NR // NSUB)
        pltpu.sync_copy(z_h.at[pl.ds(rows0, NR // NSUB)],
                        accum.at[pl.ds(rows0, NR // NSUB)])
        pltpu.sync_copy(att_h, attv)
        plsc.subcore_barrier()
        iota16 = lax.iota(jnp.int32, 16)
        NCH = EW // KE

        cbase = wid * NCH * 2 * KE
        pltpu.sync_copy(sd_h.at[pl.ds(cbase, 2 * KE)], sdv)
        pltpu.async_copy(xl2_h.at[sdv.at[pl.ds(0, KE)]], lrows, sem)
        pltpu.async_copy(xr_h.at[sdv.at[pl.ds(KE, KE)]], rrows, sem)

        def chunk(i, _):
            pltpu.make_async_copy(xl2_h.at[pl.ds(0, KE)], lrows, sem).wait()
            pltpu.make_async_copy(xr_h.at[pl.ds(0, KE)], rrows, sem).wait()
            for gc in range(KE // 16):
                dv = sdv[pl.ds(KE + gc * 16, 16)]
                scidx[pl.ds(gc * 16, 16)] = dv
                scidx[pl.ds(KE + gc * 16, 16)] = (
                    NP + lax.shift_right_arithmetic(dv, 7))
                didxe[pl.ds(gc * 16, 16)] = dv

            def group(gc, _):
                def edge16(j, lvec):
                    e = gc * 16 + j
                    acc = jnp.zeros((16,), jnp.float32)
                    for kc in range(D // 16):
                        vl = lrows[e, pl.ds(kc * 16, 16)]
                        vr = rrows[e, pl.ds(kc * 16, 16)]
                        acc = acc + jnp.abs(vl + vr) * attv[pl.ds(kc * 16, 16)]
                    bias = lrows[e, pl.ds(D, 16)][0]
                    return jnp.where(iota16 == j, _sum16(acc) + bias, lvec)

                lvec = lax.fori_loop(0, 16, edge16,
                                     jnp.zeros((16,), jnp.float32),
                                     unroll=4)
                exv[pl.ds(gc * 16, 16)] = jnp.exp(lvec)
                return 0

            lax.fori_loop(0, KE // 16, group, 0)

            def scale(e, _):
                sxv = exv[pl.ds(e, 16)][0]
                colv = didxe[pl.ds(e, 16)][0] % 128
                for kc in range(D // 16):
                    srow2[e, pl.ds(kc * 16, 16)] = (
                        lrows[e, pl.ds(kc * 16, 16)] * sxv)
                    srow2[KE + e, pl.ds(kc * 16, 16)] = jnp.where(
                        iota16 + kc * 16 == colv, sxv, 0.0)
                return 0

            lax.fori_loop(0, KE, scale, 0, unroll=4)

            @pl.when(i + 1 < NCH)
            def _():
                base = cbase + (i + 1) * 2 * KE
                pltpu.sync_copy(sd_h.at[pl.ds(base, 2 * KE)], sdv)
                pltpu.async_copy(xl2_h.at[sdv.at[pl.ds(0, KE)]], lrows, sem)
                pltpu.async_copy(xr_h.at[sdv.at[pl.ds(KE, KE)]], rrows, sem)

            pltpu.sync_copy(srow2, accum.at[scidx], add=True)
            return 0

        lax.fori_loop(0, NCH, chunk, 0)
        plsc.subcore_barrier()
        pltpu.sync_copy(accum.at[pl.ds(rows0, NR // NSUB)],
                        acc_o.at[cid, pl.ds(rows0, NR // NSUB)])

    return k(xl2, xr, att, sd, z)


def _sc_edge_sage(x2, sd, z):
    @functools.partial(
        pl.kernel, mesh=_sc_mesh(),
        out_type=jax.ShapeDtypeStruct((NSC, NR, D), jnp.float32),
        scratch_types=[pltpu.VMEM_SHARED((NR, D), jnp.float32),
                       pltpu.VMEM((2 * KE,), jnp.int32),
                       pltpu.VMEM((2 * KE,), jnp.int32),
                       pltpu.VMEM((KE + 16,), jnp.int32),
                       pltpu.VMEM((KE, 2 * D), jnp.float32),
                       pltpu.VMEM((2 * KE, D), jnp.float32),
                       pltpu.SemaphoreType.DMA],
    )
    def k(x2_h, sd_h, z_h, acc_o,
          accum, sdv, scidx, didxe, lrows, srow2, sem):
        cid = lax.axis_index("c")
        sid = lax.axis_index("s")
        wid = cid * NSUB + sid
        rows0 = sid * (NR // NSUB)
        pltpu.sync_copy(z_h.at[pl.ds(rows0, NR // NSUB)],
                        accum.at[pl.ds(rows0, NR // NSUB)])
        plsc.subcore_barrier()
        iota16 = lax.iota(jnp.int32, 16)
        NCH = EW // KE

        cbase = wid * NCH * 2 * KE
        pltpu.sync_copy(sd_h.at[pl.ds(cbase, 2 * KE)], sdv)
        pltpu.async_copy(x2_h.at[sdv.at[pl.ds(0, KE)]], lrows, sem)

        def chunk(i, _):
            pltpu.make_async_copy(x2_h.at[pl.ds(0, KE)], lrows, sem).wait()
            for gc in range(KE // 16):
                dv = sdv[pl.ds(KE + gc * 16, 16)]
                scidx[pl.ds(gc * 16, 16)] = dv
                scidx[pl.ds(KE + gc * 16, 16)] = (
                    NP + lax.shift_right_arithmetic(dv, 7))
                didxe[pl.ds(gc * 16, 16)] = dv

            def scale(e, _):
                sxv = lrows[e, pl.ds(D, 16)][0]
                colv = didxe[pl.ds(e, 16)][0] % 128
                for kc in range(D // 16):
                    srow2[e, pl.ds(kc * 16, 16)] = lrows[e, pl.ds(kc * 16, 16)]
                    srow2[KE + e, pl.ds(kc * 16, 16)] = jnp.where(
                        iota16 + kc * 16 == colv, sxv, 0.0)
                return 0

            lax.fori_loop(0, KE, scale, 0, unroll=4)

            @pl.when(i + 1 < NCH)
            def _():
                base = cbase + (i + 1) * 2 * KE
                pltpu.sync_copy(sd_h.at[pl.ds(base, 2 * KE)], sdv)
                pltpu.async_copy(x2_h.at[sdv.at[pl.ds(0, KE)]], lrows, sem)

            pltpu.sync_copy(srow2, accum.at[scidx], add=True)
            return 0

        lax.fori_loop(0, NCH, chunk, 0)
        plsc.subcore_barrier()
        pltpu.sync_copy(accum.at[pl.ds(rows0, NR // NSUB)],
                        acc_o.at[cid, pl.ds(rows0, NR // NSUB)])

    return k(x2, sd, z)


def _pool(x, bcol, bsm, alive, ordk, w):
    score, tab = _score_tables(x, w, bcol, alive)
    tsc3, al3, ord3 = _rank(score, bcol, alive, ordk, tab, bsm)
    tsc = tsc3.reshape(NP, 1)
    alive = al3.reshape(NP, 1)
    ordk = ord3.reshape(NP, 1)
    x = _scale_rows(x, tsc)
    r = _gpool(x, alive, tab)
    return x, alive, ordk, r


def _branch(x0, ei, batch, P, c1, p1, c2, p2, s3, p3, z):
    epad = jnp.full((2, EP - E), N, jnp.int32)
    sd2 = jnp.concatenate([ei.astype(jnp.int32), epad], axis=1)
    sd = jnp.concatenate([sd2[0].reshape(-1, KE),
                          sd2[1].reshape(-1, KE)], axis=1).reshape(-1)
    bp = jnp.concatenate([batch.astype(jnp.int32),
                          jnp.full((NP - N,), G, jnp.int32)])
    bcol = bp.reshape(NP, 1)
    bsm = bp.reshape(NB, 128)
    x = jnp.pad(x0, ((0, NP - N), (0, 0)))
    alive = jnp.concatenate([jnp.ones((N, 1), jnp.float32),
                             jnp.zeros((NP - N, 1), jnp.float32)])
    ordk = jnp.arange(NP, dtype=jnp.float32).reshape(NP, 1)

    xl2, xr, exs = _gat_pre(x, alive, P[c1])
    acc = _sc_edge_gat(xl2, xr, 0.4 * P[c1]['att'], sd, z)
    x = _gat_post(acc[0, :NP], acc[1, :NP],
                  acc[0, NP:NP + NB].reshape(NP, 1), acc[1, NP:NP + NB].reshape(NP, 1),
                  xl2[:, :D], exs, P[c1]['b'], alive)
    x, alive, ordk, r1 = _pool(x, bcol, bsm, alive, ordk, P[p1])

    xl2, xr, exs = _gat_pre(x, alive, P[c2])
    acc = _sc_edge_gat(xl2, xr, 0.4 * P[c2]['att'], sd, z)
    x = _gat_post(acc[0, :NP], acc[1, :NP],
                  acc[0, NP:NP + NB].reshape(NP, 1), acc[1, NP:NP + NB].reshape(NP, 1),
                  xl2[:, :D], exs, P[c2]['b'], alive)
    x, alive, ordk, r2 = _pool(x, bcol, bsm, alive, ordk, P[p2])

    x2 = _sage_pre(x, alive)
    acc = _sc_edge_sage(x2, sd, z)
    x = _sage_post(acc[0, :NP], acc[1, :NP],
                   acc[0, NP:NP + NB].reshape(NP, 1), acc[1, NP:NP + NB].reshape(NP, 1),
                   x, P[s3], alive)
    x, alive, ordk, r3 = _pool(x, bcol, bsm, alive, ordk, P[p3])
    return r1, r2, r3


def kernel(source_x, source_edge_index, source_batch,
           target_x, target_edge_index, target_batch, params):
    z = jnp.zeros((NR, D), jnp.float32)
    s1, s2, s3 = _branch(source_x, source_edge_index, source_batch, params,
                         'c11', 'p11', 'c12', 'p12', 's13', 'p13', z)
    z2, _ = lax.optimization_barrier((z, s3))
    t1, t2, t3 = _branch(target_x, target_edge_index, target_batch, params,
                         'c21', 'p21', 'c22', 'p22', 's23', 'p23', z2)
    out = _mlp(s1, s2, s3, t1, t2, t3, params)
    return out[:, :2]


# speedup vs baseline: 1.0317x; 1.0317x over previous
import functools

import jax
import jax.numpy as jnp
from jax import lax
from jax.experimental import pallas as pl
from jax.experimental.pallas import tpu as pltpu
from jax.experimental.pallas import tpu_sc as plsc

N = 10000
E = 320000
D = 128
G = 64
NP = 10240
NB = NP // 128
NR = 10368
RB = 512
NG = NP // RB
NSC = 2
NSUB = 16
NW = NSC * NSUB
KE = 64
EP = 321536
EW = EP // NW
NEG = -3.0e38


def _leaky(x, s):
    return jnp.where(x > 0, x, s * x)


def _eye128():
    a = lax.broadcasted_iota(jnp.int32, (128, 128), 0)
    b = lax.broadcasted_iota(jnp.int32, (128, 128), 1)
    return (a == b).astype(jnp.float32)


def _k_gat_pre(x_ref, a_ref, wl_ref, bl_ref, wr_ref, br_ref, att_ref,
               xl2_ref, xr_ref, exs_ref):
    x = x_ref[...]
    att = att_ref[...]
    xl = jnp.dot(x, wl_ref[...], preferred_element_type=jnp.float32) + bl_ref[...]
    xr = jnp.dot(x, wr_ref[...], preferred_element_type=jnp.float32) + br_ref[...]
    xl2_ref[:, :D] = xl
    lane0 = (lax.broadcasted_iota(jnp.int32, (1, D), 1) == 0).astype(jnp.float32)
    al = jnp.sum(xl * att, axis=1, keepdims=True)
    bias = (a_ref[...] - 1.0) * 1e4 + 0.6 * al
    xl2_ref[:, D:] = bias * lane0
    xr_ref[...] = xr
    t = xl + xr
    s = 0.6 * al + 0.4 * jnp.sum(jnp.abs(t) * att, axis=1, keepdims=True)
    exs_ref[...] = jnp.broadcast_to(jnp.exp(s), (RB, D))


def _gat_pre(x, alive, p):
    f = pl.pallas_call(
        _k_gat_pre,
        grid=(NG,),
        in_specs=[pl.BlockSpec((RB, D), lambda i: (i, 0)),
                  pl.BlockSpec((RB, 1), lambda i: (i, 0)),
                  pl.BlockSpec((D, D), lambda i: (0, 0)),
                  pl.BlockSpec((1, D), lambda i: (0, 0)),
                  pl.BlockSpec((D, D), lambda i: (0, 0)),
                  pl.BlockSpec((1, D), lambda i: (0, 0)),
                  pl.BlockSpec((1, D), lambda i: (0, 0))],
        out_specs=[pl.BlockSpec((RB, 2 * D), lambda i: (i, 0)),
                   pl.BlockSpec((RB, D), lambda i: (i, 0)),
                   pl.BlockSpec((RB, D), lambda i: (i, 0))],
        out_shape=[jax.ShapeDtypeStruct((NP, 2 * D), jnp.float32),
                   jax.ShapeDtypeStruct((NP, D), jnp.float32),
                   jax.ShapeDtypeStruct((NP, D), jnp.float32)],
    )
    return f(x, alive, p['Wl'], p['bl'].reshape(1, D), p['Wr'],
             p['br'].reshape(1, D), p['att'].reshape(1, D))


def _k_gat_post(a0_ref, a1_ref, d0_ref, d1_ref, xl_ref, exs_ref, b_ref,
                al_ref, o_ref):
    num = a0_ref[...] + a1_ref[...]
    den = d0_ref[...] + d1_ref[...]
    exs = exs_ref[:, 0:1]
    num = num + exs * xl_ref[...]
    den = den + exs
    o = num / (den + 1e-16) + b_ref[...]
    o_ref[...] = _leaky(o, 0.01) * al_ref[...]


def _gat_post(a0, a1, d0, d1, xl, exs, b, alive):
    f = pl.pallas_call(
        _k_gat_post,
        grid=(NG,),
        in_specs=[pl.BlockSpec((RB, D), lambda i: (i, 0)),
                  pl.BlockSpec((RB, D), lambda i: (i, 0)),
                  pl.BlockSpec((RB, 1), lambda i: (i, 0)),
                  pl.BlockSpec((RB, 1), lambda i: (i, 0)),
                  pl.BlockSpec((RB, D), lambda i: (i, 0)),
                  pl.BlockSpec((RB, D), lambda i: (i, 0)),
                  pl.BlockSpec((1, D), lambda i: (0, 0)),
                  pl.BlockSpec((RB, 1), lambda i: (i, 0))],
        out_specs=pl.BlockSpec((RB, D), lambda i: (i, 0)),
        out_shape=jax.ShapeDtypeStruct((NP, D), jnp.float32),
    )
    return f(a0, a1, d0, d1, xl, exs, b.reshape(1, D), alive)


def _k_sage_pre(x_ref, a_ref, o_ref):
    o_ref[:, :D] = x_ref[...]
    lane0 = (lax.broadcasted_iota(jnp.int32, (1, D), 1) == 0).astype(jnp.float32)
    o_ref[:, D:] = a_ref[...] * lane0


def _sage_pre(x, alive):
    f = pl.pallas_call(
        _k_sage_pre,
        grid=(NG,),
        in_specs=[pl.BlockSpec((RB, D), lambda i: (i, 0)),
                  pl.BlockSpec((RB, 1), lambda i: (i, 0))],
        out_specs=pl.BlockSpec((RB, 2 * D), lambda i: (i, 0)),
        out_shape=jax.ShapeDtypeStruct((NP, 2 * D), jnp.float32),
    )
    return f(x, alive)


def _k_sage_post(a0_ref, a1_ref, d0_ref, d1_ref, x_ref, wl_ref, wr_ref,
                 bl_ref, al_ref, o_ref):
    num = a0_ref[...] + a1_ref[...]
    cnt = d0_ref[...] + d1_ref[...]
    mean = num / jnp.maximum(cnt, 1.0)
    o = (jnp.dot(mean, wl_ref[...], preferred_element_type=jnp.float32)
         + jnp.dot(x_ref[...], wr_ref[...], preferred_element_type=jnp.float32)
         + bl_ref[...])
    o_ref[...] = _leaky(o, 0.01) * al_ref[...]


def _sage_post(a0, a1, d0, d1, x, p, alive):
    f = pl.pallas_call(
        _k_sage_post,
        grid=(NG,),
        in_specs=[pl.BlockSpec((RB, D), lambda i: (i, 0)),
                  pl.BlockSpec((RB, D), lambda i: (i, 0)),
                  pl.BlockSpec((RB, 1), lambda i: (i, 0)),
                  pl.BlockSpec((RB, 1), lambda i: (i, 0)),
                  pl.BlockSpec((RB, D), lambda i: (i, 0)),
                  pl.BlockSpec((D, D), lambda i: (0, 0)),
                  pl.BlockSpec((D, D), lambda i: (0, 0)),
                  pl.BlockSpec((1, D), lambda i: (0, 0)),
                  pl.BlockSpec((RB, 1), lambda i: (i, 0))],
        out_specs=pl.BlockSpec((RB, D), lambda i: (i, 0)),
        out_shape=jax.ShapeDtypeStruct((NP, D), jnp.float32),
    )
    return f(a0, a1, d0, d1, x, p['Wl'], p['Wr'], p['bl'].reshape(1, D), alive)


def _k_score_tables(x_ref, w_ref, bcol_ref, acol_ref, score_ref, tab_ref):
    w = w_ref[...]
    rnorm = 1.0 / (jnp.sqrt(jnp.sum(w * w)) + 1e-16)
    g_row = lax.broadcasted_iota(jnp.int32, (1, 128), 1).astype(jnp.float32)
    counts_a = jnp.zeros((1, 128), jnp.float32)
    counts_o = jnp.zeros((1, 128), jnp.float32)
    starts = jnp.zeros((1, 128), jnp.float32)
    for r in range(NB):
        rows = x_ref[pl.ds(r * 128, 128), :]
        s = jnp.sum(rows * w, axis=1, keepdims=True)
        score_ref[pl.ds(r * 128, 128), :] = jnp.tanh(s * rnorm)
        bcol = bcol_ref[pl.ds(r * 128, 128), :].astype(jnp.float32)
        acol = acol_ref[pl.ds(r * 128, 128), :]
        eqg = (bcol == g_row).astype(jnp.float32)
        counts_a = counts_a + jnp.sum(eqg * acol, axis=0, keepdims=True)
        counts_o = counts_o + jnp.sum(eqg, axis=0, keepdims=True)
        starts = starts + jnp.sum((bcol < g_row).astype(jnp.float32),
                                  axis=0, keepdims=True)
    kk = jnp.where(g_row < G, jnp.floor((counts_a + 1.0) * 0.5), 0.0)
    lt = (lax.broadcasted_iota(jnp.int32, (128, 128), 0)
          < lax.broadcasted_iota(jnp.int32, (128, 128), 1)).astype(jnp.float32)
    prefix_kk = jnp.dot(kk, lt, preferred_element_type=jnp.float32)
    tab_ref[...] = jnp.concatenate(
        [kk, starts, counts_o, prefix_kk,
         jnp.zeros((4, 128), jnp.float32)], axis=0)


def _score_tables(x, w, bcol, acol):
    f = pl.pallas_call(
        _k_score_tables,
        grid=(1,),
        in_specs=[pl.BlockSpec((NP, D), lambda i: (0, 0)),
                  pl.BlockSpec((1, D), lambda i: (0, 0)),
                  pl.BlockSpec((NP, 1), lambda i: (0, 0)),
                  pl.BlockSpec((NP, 1), lambda i: (0, 0))],
        out_specs=[pl.BlockSpec((NP, 1), lambda i: (0, 0)),
                   pl.BlockSpec((8, 128), lambda i: (0, 0))],
        out_shape=[jax.ShapeDtypeStruct((NP, 1), jnp.float32),
                   jax.ShapeDtypeStruct((8, 128), jnp.float32)],
    )
    return f(x, w.reshape(1, D), bcol, acol)


def _k_rank(score_ref, bcol_ref, acol_ref, ocol_ref, tab_ref, tabs_ref,
            bsm_ref, tsc_ref, al_ref, ord_ref):
    r = pl.program_id(0)
    eye = _eye128()
    g_col = lax.broadcasted_iota(jnp.int32, (128, 1), 0).astype(jnp.float32)
    scol = score_ref[pl.ds(r * 128, 128), :]
    bcolf = bcol_ref[pl.ds(r * 128, 128), :].astype(jnp.float32)
    acolf = acol_ref[pl.ds(r * 128, 128), :]
    ocolf = ocol_ref[pl.ds(r * 128, 128), :]
    si = jnp.sum(eye * scol, axis=0, keepdims=True)
    bi = jnp.sum(eye * bcolf, axis=0, keepdims=True)
    ai = jnp.sum(eye * acolf, axis=0, keepdims=True)
    oi = jnp.sum(eye * ocolf, axis=0, keepdims=True)

    bmin = bsm_ref[r, 0]
    bmax = bsm_ref[r, 127]
    c_lo = tabs_ref[1, bmin].astype(jnp.int32)
    c_hi = (tabs_ref[1, bmax] + tabs_ref[2, bmax]).astype(jnp.int32)
    t_lo = c_lo // 128
    t_hi = (c_hi + 127) // 128

    def body(t, rank):
        sj = score_ref[pl.ds(t * 128, 128), :]
        bj = bcol_ref[pl.ds(t * 128, 128), :].astype(jnp.float32)
        aj = acol_ref[pl.ds(t * 128, 128), :]
        oj = ocol_ref[pl.ds(t * 128, 128), :]
        beats = (sj > si) | ((sj == si) & (oj < oi))
        cmp = (bj == bi) & beats & (aj > 0)
        return rank + jnp.sum(cmp.astype(jnp.float32), axis=0, keepdims=True)

    rank = lax.fori_loop(t_lo, t_hi, body, jnp.zeros((1, 128), jnp.float32))

    kk_col = jnp.sum(eye * tab_ref[0:1, :], axis=1, keepdims=True)
    pf_col = jnp.sum(eye * tab_ref[3:4, :], axis=1, keepdims=True)
    bmask = (g_col == bi).astype(jnp.float32)
    kk_b = jnp.sum(bmask * kk_col, axis=0, keepdims=True)
    pf_b = jnp.sum(bmask * pf_col, axis=0, keepdims=True)
    kept = (ai > 0) & (rank < kk_b)
    tsc_ref[...] = jnp.where(kept, si, 0.0).reshape(1, 1, 128)
    al_ref[...] = jnp.where(kept, 1.0, 0.0).reshape(1, 1, 128)
    ord_ref[...] = jnp.where(kept, pf_b + rank, 0.0).reshape(1, 1, 128)


def _rank(score, bcol, acol, ocol, tab, bsm):
    f = pl.pallas_call(
        _k_rank,
        grid=(NB,),
        in_specs=[pl.BlockSpec((NP, 1), lambda i: (0, 0)),
                  pl.BlockSpec((NP, 1), lambda i: (0, 0)),
                  pl.BlockSpec((NP, 1), lambda i: (0, 0)),
                  pl.BlockSpec((NP, 1), lambda i: (0, 0)),
                  pl.BlockSpec((8, 128), lambda i: (0, 0)),
                  pl.BlockSpec(memory_space=pltpu.SMEM),
                  pl.BlockSpec(memory_space=pltpu.SMEM)],
        out_specs=[pl.BlockSpec((1, 1, 128), lambda i: (i, 0, 0))] * 3,
        out_shape=[jax.ShapeDtypeStruct((NB, 1, 128), jnp.float32)] * 3,
    )
    return f(score, bcol, acol, ocol, tab, tab, bsm)


def _k_scale_rows(x_ref, t_ref, o_ref):
    o_ref[...] = x_ref[...] * t_ref[...]


def _scale_rows(x, tsc):
    f = pl.pallas_call(
        _k_scale_rows,
        grid=(NG,),
        in_specs=[pl.BlockSpec((RB, D), lambda i: (i, 0)),
                  pl.BlockSpec((RB, 1), lambda i: (i, 0))],
        out_specs=pl.BlockSpec((RB, D), lambda i: (i, 0)),
        out_shape=jax.ShapeDtypeStruct((NP, D), jnp.float32),
    )
    return f(x, tsc)


def _k_gpool(x_ref, acol_ref, tabs_ref, o_ref):
    pid = pl.program_id(0)
    sub = lax.broadcasted_iota(jnp.int32, (8, 1), 0).astype(jnp.float32)
    for gg in range(8):
        g = pid * 8 + gg
        kkg = tabs_ref[0, g].astype(jnp.int32)
        s = tabs_ref[1, g].astype(jnp.int32)
        co = tabs_ref[2, g].astype(jnp.int32)
        nch = (co + 7) // 8

        def body(t, carry):
            msum, mmax = carry
            rows = x_ref[pl.ds(s + t * 8, 8), :]
            am = acol_ref[pl.ds(s + t * 8, 8), :]
            pos = (s + t * 8).astype(jnp.float32) + sub
            rmask = (pos < (s + co).astype(jnp.float32)) & (am > 0)
            msum = msum + jnp.where(rmask, rows, 0.0)
            mmax = jnp.maximum(mmax, jnp.where(rmask, rows, NEG))
            return msum, mmax

        msum, mmax = lax.fori_loop(
            0, nch, body,
            (jnp.zeros((8, D), jnp.float32), jnp.full((8, D), NEG, jnp.float32)))
        colsum = jnp.sum(msum, axis=0, keepdims=True)
        colmax = jnp.max(mmax, axis=0, keepdims=True)
        kf = kkg.astype(jnp.float32)
        mean = colsum / jnp.maximum(kf, 1.0)
        mx = jnp.where(kkg > 0, colmax, 0.0)
        o_ref[pl.ds(gg, 1), 0:D] = mx
        o_ref[pl.ds(gg, 1), D:2 * D] = mean


def _gpool(x, acol, tab):
    f = pl.pallas_call(
        _k_gpool,
        grid=(8,),
        in_specs=[pl.BlockSpec((NP, D), lambda i: (0, 0)),
                  pl.BlockSpec((NP, 1), lambda i: (0, 0)),
                  pl.BlockSpec(memory_space=pltpu.SMEM)],
        out_specs=pl.BlockSpec((8, 2 * D), lambda i: (i, 0)),
        out_shape=jax.ShapeDtypeStruct((G, 2 * D), jnp.float32),
    )
    return f(x, acol, tab)


def _k_mlp(s1, s2, s3, t1, t2, t3, w1a, w1b, b1, w2, b2, w3, b3, o_ref):
    s = s1[...] + s2[...] + s3[...]
    t = t1[...] + t2[...] + t3[...]
    h = (jnp.dot(s, w1a[...], preferred_element_type=jnp.float32)
         + jnp.dot(t, w1b[...], preferred_element_type=jnp.float32) + b1[...])
    h = _leaky(h, 0.01)
    h = _leaky(jnp.dot(h, w2[...], preferred_element_type=jnp.float32) + b2[...], 0.01)
    lg = jnp.dot(h, w3[...], preferred_element_type=jnp.float32) + b3[...]
    lanemask = lax.broadcasted_iota(jnp.int32, (1, 128), 1) < 2
    ex = jnp.where(lanemask, jnp.exp(lg), 0.0)
    ssum = jnp.sum(ex, axis=1, keepdims=True)
    o_ref[...] = lg - jnp.log(ssum)


def _mlp(s1, s2, s3, t1, t2, t3, P):
    w1 = P['lin1']['W']
    w2 = jnp.pad(P['lin2']['W'], ((0, 0), (0, 64)))
    b2 = jnp.pad(P['lin2']['b'], (0, 64)).reshape(1, 128)
    w3 = jnp.pad(P['lin3']['W'], ((0, 64), (0, 126)))
    b3 = jnp.pad(P['lin3']['b'], (0, 126)).reshape(1, 128)
    f = pl.pallas_call(
        _k_mlp,
        grid=(1,),
        in_specs=[pl.BlockSpec((G, 2 * D), lambda i: (0, 0))] * 6
        + [pl.BlockSpec((2 * D, D), lambda i: (0, 0)),
           pl.BlockSpec((2 * D, D), lambda i: (0, 0)),
           pl.BlockSpec((1, D), lambda i: (0, 0)),
           pl.BlockSpec((D, D), lambda i: (0, 0)),
           pl.BlockSpec((1, D), lambda i: (0, 0)),
           pl.BlockSpec((D, D), lambda i: (0, 0)),
           pl.BlockSpec((1, D), lambda i: (0, 0))],
        out_specs=pl.BlockSpec((G, D), lambda i: (0, 0)),
        out_shape=jax.ShapeDtypeStruct((G, D), jnp.float32),
    )
    return f(s1, s2, s3, t1, t2, t3, w1[:256], w1[256:],
             P['lin1']['b'].reshape(1, D), w2, b2, w3, b3)


def _sum16(v):
    for b in range(4):
        idx = lax.iota(jnp.int32, 16) ^ (1 << b)
        v = v + v.at[idx].get(mode='promise_in_bounds')
    return v


def _sc_mesh():
    return plsc.VectorSubcoreMesh(core_axis_name="c", subcore_axis_name="s",
                                  num_cores=NSC, num_subcores=NSUB)


def _sc_edge_gat(xl2, xr, att, sd, z):
    @functools.partial(
        pl.kernel, mesh=_sc_mesh(),
        out_type=jax.ShapeDtypeStruct((NSC, NR, D), jnp.float32),
        scratch_types=[pltpu.VMEM_SHARED((NR, D), jnp.float32),
                       pltpu.VMEM((2 * KE,), jnp.int32),
                       pltpu.VMEM((2 * KE,), jnp.int32),
                       pltpu.VMEM((KE + 16,), jnp.int32),
                       pltpu.VMEM((KE, 2 * D), jnp.float32),
                       pltpu.VMEM((KE, D), jnp.float32),
                       pltpu.VMEM((D,), jnp.float32),
                       pltpu.VMEM((2 * KE, D), jnp.float32),
                       pltpu.VMEM((KE + 16,), jnp.float32),
                       pltpu.SemaphoreType.DMA],
    )
    def k(xl2_h, xr_h, att_h, sd_h, z_h, acc_o,
          accum, sdv, scidx, didxe, lrows, rrows, attv, srow2, exv, sem):
        cid = lax.axis_index("c")
        sid = lax.axis_index("s")
        wid = cid * NSUB + sid
        rows0 = sid * (NR // NSUB)
        pltpu.sync_copy(z_h.at[pl.ds(rows0, NR // NSUB)],
                        accum.at[pl.ds(rows0, NR // NSUB)])
        pltpu.sync_copy(att_h, attv)
        plsc.subcore_barrier()
        iota16 = lax.iota(jnp.int32, 16)
        NCH = EW // KE

        cbase = wid * NCH * 2 * KE
        pltpu.sync_copy(sd_h.at[pl.ds(cbase, 2 * KE)], sdv)
        pltpu.async_copy(xl2_h.at[sdv.at[pl.ds(0, KE)]], lrows, sem)
        pltpu.async_copy(xr_h.at[sdv.at[pl.ds(KE, KE)]], rrows, sem)

        def chunk(i, _):
            pltpu.make_async_copy(xl2_h.at[pl.ds(0, KE)], lrows, sem).wait()
            pltpu.make_async_copy(xr_h.at[pl.ds(0, KE)], rrows, sem).wait()
            for gc in range(KE // 16):
                dv = sdv[pl.ds(KE + gc * 16, 16)]
                scidx[pl.ds(gc * 16, 16)] = dv
                scidx[pl.ds(KE + gc * 16, 16)] = (
                    NP + lax.shift_right_arithmetic(dv, 7))
                didxe[pl.ds(gc * 16, 16)] = dv

            def group(gc, _):
                def edge16(j, lvec):
                    e = gc * 16 + j
                    acc = jnp.zeros((16,), jnp.float32)
                    for kc in range(D // 16):
                        vl = lrows[e, pl.ds(kc * 16, 16)]
                        vr = rrows[e, pl.ds(kc * 16, 16)]
                        acc = acc + jnp.abs(vl + vr) * attv[pl.ds(kc * 16, 16)]
                    bias = lrows[e, pl.ds(D, 16)][0]
                    return jnp.where(iota16 == j, _sum16(acc) + bias, lvec)

                lvec = lax.fori_loop(0, 16, edge16,
                                     jnp.zeros((16,), jnp.float32))
                exv[pl.ds(gc * 16, 16)] = jnp.exp(lvec)
                return 0

            lax.fori_loop(0, KE // 16, group, 0)

            def scale(e, _):
                sxv = exv[pl.ds(e, 16)][0]
                colv = didxe[pl.ds(e, 16)][0] % 128
                for kc in range(D // 16):
                    srow2[e, pl.ds(kc * 16, 16)] = (
                        lrows[e, pl.ds(kc * 16, 16)] * sxv)
                    srow2[KE + e, pl.ds(kc * 16, 16)] = jnp.where(
                        iota16 + kc * 16 == colv, sxv, 0.0)
                return 0

            lax.fori_loop(0, KE, scale, 0)

            @pl.when(i + 1 < NCH)
            def _():
                base = cbase + (i + 1) * 2 * KE
                pltpu.sync_copy(sd_h.at[pl.ds(base, 2 * KE)], sdv)
                pltpu.async_copy(xl2_h.at[sdv.at[pl.ds(0, KE)]], lrows, sem)
                pltpu.async_copy(xr_h.at[sdv.at[pl.ds(KE, KE)]], rrows, sem)

            pltpu.sync_copy(srow2, accum.at[scidx], add=True)
            return 0

        lax.fori_loop(0, NCH, chunk, 0)
        plsc.subcore_barrier()
        pltpu.sync_copy(accum.at[pl.ds(rows0, NR // NSUB)],
                        acc_o.at[cid, pl.ds(rows0, NR // NSUB)])

    return k(xl2, xr, att, sd, z)


def _sc_edge_sage(x2, sd, z):
    @functools.partial(
        pl.kernel, mesh=_sc_mesh(),
        out_type=jax.ShapeDtypeStruct((NSC, NR, D), jnp.float32),
        scratch_types=[pltpu.VMEM_SHARED((NR, D), jnp.float32),
                       pltpu.VMEM((2 * KE,), jnp.int32),
                       pltpu.VMEM((2 * KE,), jnp.int32),
                       pltpu.VMEM((KE + 16,), jnp.int32),
                       pltpu.VMEM((KE, 2 * D), jnp.float32),
                       pltpu.VMEM((2 * KE, D), jnp.float32),
                       pltpu.SemaphoreType.DMA],
    )
    def k(x2_h, sd_h, z_h, acc_o,
          accum, sdv, scidx, didxe, lrows, srow2, sem):
        cid = lax.axis_index("c")
        sid = lax.axis_index("s")
        wid = cid * NSUB + sid
        rows0 = sid * (NR // NSUB)
        pltpu.sync_copy(z_h.at[pl.ds(rows0, NR // NSUB)],
                        accum.at[pl.ds(rows0, NR // NSUB)])
        plsc.subcore_barrier()
        iota16 = lax.iota(jnp.int32, 16)
        NCH = EW // KE

        cbase = wid * NCH * 2 * KE
        pltpu.sync_copy(sd_h.at[pl.ds(cbase, 2 * KE)], sdv)
        pltpu.async_copy(x2_h.at[sdv.at[pl.ds(0, KE)]], lrows, sem)

        def chunk(i, _):
            pltpu.make_async_copy(x2_h.at[pl.ds(0, KE)], lrows, sem).wait()
            for gc in range(KE // 16):
                dv = sdv[pl.ds(KE + gc * 16, 16)]
                scidx[pl.ds(gc * 16, 16)] = dv
                scidx[pl.ds(KE + gc * 16, 16)] = (
                    NP + lax.shift_right_arithmetic(dv, 7))
                didxe[pl.ds(gc * 16, 16)] = dv

            def scale(e, _):
                sxv = lrows[e, pl.ds(D, 16)][0]
                colv = didxe[pl.ds(e, 16)][0] % 128
                for kc in range(D // 16):
                    srow2[e, pl.ds(kc * 16, 16)] = lrows[e, pl.ds(kc * 16, 16)]
                    srow2[KE + e, pl.ds(kc * 16, 16)] = jnp.where(
                        iota16 + kc * 16 == colv, sxv, 0.0)
                return 0

            lax.fori_loop(0, KE, scale, 0)

            @pl.when(i + 1 < NCH)
            def _():
                base = cbase + (i + 1) * 2 * KE
                pltpu.sync_copy(sd_h.at[pl.ds(base, 2 * KE)], sdv)
                pltpu.async_copy(x2_h.at[sdv.at[pl.ds(0, KE)]], lrows, sem)

            pltpu.sync_copy(srow2, accum.at[scidx], add=True)
            return 0

        lax.fori_loop(0, NCH, chunk, 0)
        plsc.subcore_barrier()
        pltpu.sync_copy(accum.at[pl.ds(rows0, NR // NSUB)],
                        acc_o.at[cid, pl.ds(rows0, NR // NSUB)])

    return k(x2, sd, z)


def _pool(x, bcol, bsm, alive, ordk, w):
    score, tab = _score_tables(x, w, bcol, alive)
    tsc3, al3, ord3 = _rank(score, bcol, alive, ordk, tab, bsm)
    tsc = tsc3.reshape(NP, 1)
    alive = al3.reshape(NP, 1)
    ordk = ord3.reshape(NP, 1)
    x = _scale_rows(x, tsc)
    r = _gpool(x, alive, tab)
    return x, alive, ordk, r


def _branch(x0, ei, batch, P, c1, p1, c2, p2, s3, p3, z):
    epad = jnp.full((2, EP - E), N, jnp.int32)
    sd2 = jnp.concatenate([ei.astype(jnp.int32), epad], axis=1)
    sd = jnp.concatenate([sd2[0].reshape(-1, KE),
                          sd2[1].reshape(-1, KE)], axis=1).reshape(-1)
    bp = jnp.concatenate([batch.astype(jnp.int32),
                          jnp.full((NP - N,), G, jnp.int32)])
    bcol = bp.reshape(NP, 1)
    bsm = bp.reshape(NB, 128)
    x = jnp.pad(x0, ((0, NP - N), (0, 0)))
    alive = jnp.concatenate([jnp.ones((N, 1), jnp.float32),
                             jnp.zeros((NP - N, 1), jnp.float32)])
    ordk = jnp.arange(NP, dtype=jnp.float32).reshape(NP, 1)

    xl2, xr, exs = _gat_pre(x, alive, P[c1])
    acc = _sc_edge_gat(xl2, xr, 0.4 * P[c1]['att'], sd, z)
    x = _gat_post(acc[0, :NP], acc[1, :NP],
                  acc[0, NP:NP + NB].reshape(NP, 1), acc[1, NP:NP + NB].reshape(NP, 1),
                  xl2[:, :D], exs, P[c1]['b'], alive)
    x, alive, ordk, r1 = _pool(x, bcol, bsm, alive, ordk, P[p1])

    xl2, xr, exs = _gat_pre(x, alive, P[c2])
    acc = _sc_edge_gat(xl2, xr, 0.4 * P[c2]['att'], sd, z)
    x = _gat_post(acc[0, :NP], acc[1, :NP],
                  acc[0, NP:NP + NB].reshape(NP, 1), acc[1, NP:NP + NB].reshape(NP, 1),
                  xl2[:, :D], exs, P[c2]['b'], alive)
    x, alive, ordk, r2 = _pool(x, bcol, bsm, alive, ordk, P[p2])

    x2 = _sage_pre(x, alive)
    acc = _sc_edge_sage(x2, sd, z)
    x = _sage_post(acc[0, :NP], acc[1, :NP],
                   acc[0, NP:NP + NB].reshape(NP, 1), acc[1, NP:NP + NB].reshape(NP, 1),
                   x, P[s3], alive)
    x, alive, ordk, r3 = _pool(x, bcol, bsm, alive, ordk, P[p3])
    return r1, r2, r3


def kernel(source_x, source_edge_index, source_batch,
           target_x, target_edge_index, target_batch, params):
    z = jnp.zeros((NR, D), jnp.float32)
    s1, s2, s3 = _branch(source_x, source_edge_index, source_batch, params,
                         'c11', 'p11', 'c12', 'p12', 's13', 'p13', z)
    z2, _ = lax.optimization_barrier((z, s3))
    t1, t2, t3 = _branch(target_x, target_edge_index, target_batch, params,
                         'c21', 'p21', 'c22', 'p22', 's23', 'p23', z2)
    out = _mlp(s1, s2, s3, t1, t2, t3, params)
    return out[:, :2]
